# baseline (speedup 1.0000x reference)
import functools

import jax
import jax.numpy as jnp
import numpy as np
from jax import lax
from jax.experimental import pallas as pl
from jax.experimental.pallas import tpu as pltpu
from jax.experimental.pallas import tpu_sc as plsc

KS = 3


def _fused_body(cr_ref, cc_ref, a_ref, x_ref,
                w1_ref, r1_ref, b1_ref,
                w2_ref, r2_ref, b2_ref,
                w3_ref, r3_ref, b3_ref,
                wfc_ref, bfc_ref,
                logits_ref, u0_ref, u1_ref,
                ws_ref, idg_ref):
    s = pl.program_id(0)
    par = lax.rem(s, 2)
    a = a_ref[0]
    cx_row = cr_ref[0, 0:1, :]
    cy_row = cr_ref[0, 1:2, :]
    cx_col = cc_ref[0, :, 0:1]
    cy_col = cc_ref[0, :, 1:2]

    dx = cx_row - cx_col
    dy = cy_row - cy_col
    rho = jnp.sqrt(dx * dx + dy * dy + 1e-12)
    theta = jnp.arctan2(dy, dx)
    mask = (a > 0).astype(jnp.float32)
    rho_max = jnp.max(rho * mask) + 1e-12
    u0 = (rho / rho_max) * mask
    u1 = (theta * np.float32(1.0 / (2.0 * np.pi)) + 0.5) * mask
    u0_ref[0] = u0
    u1_ref[0] = u1

    p0 = u0 * (KS - 1)
    p1 = u1 * (KS - 1)
    b0 = [jnp.maximum(0.0, 1.0 - jnp.abs(p0 - k)) for k in range(KS)]
    b1 = [jnp.maximum(0.0, 1.0 - jnp.abs(p1 - k)) for k in range(KS)]

    deg = jnp.sum(a, axis=1, keepdims=True)
    idg_ref[par] = 1.0 / jnp.maximum(deg, 1.0)

    for k1 in range(KS):
        ab0 = a * b0[k1]
        for k2 in range(KS):
            ws_ref[par, k1 * KS + k2] = (ab0 * b1[k2]).astype(jnp.bfloat16)

    pp = lax.rem(s + 1, 2)
    inv_deg = idg_ref[pp]

    def mm(p, q):
        return jax.lax.dot_general(
            p, q, (((1,), (0,)), ((), ())),
            preferred_element_type=jnp.float32,
            precision=jax.lax.Precision.DEFAULT)

    def layer(h, w_ref, r_ref, b_ref):
        y = mm(h, w_ref[...])
        out = mm(h, r_ref[...]) + b_ref[...]
        c = r_ref.shape[1]
        msg_a = jnp.zeros_like(out)
        msg_b = jnp.zeros_like(out)
        yb = y.astype(jnp.bfloat16)
        for kidx in range(KS * KS):
            m = mm(ws_ref[pp, kidx], yb[:, kidx * c:(kidx + 1) * c])
            if kidx % 2 == 0:
                msg_a = msg_a + m
            else:
                msg_b = msg_b + m
        return jax.nn.relu(out + (msg_a + msg_b) * inv_deg)

    h = layer(x_ref[0], w1_ref, r1_ref, b1_ref)
    h = layer(h, w2_ref, r2_ref, b2_ref)
    h = layer(h, w3_ref, r3_ref, b3_ref)

    g = jnp.max(h, axis=0, keepdims=True)
    logits_ref[0] = mm(g, wfc_ref[...]) + bfc_ref[...]


def kernel(x, A, coord, W1, R1, b1, W2, R2, b2, W3, R3, b3, Wfc, bfc):
    B, N, Cin = x.shape
    C = R1.shape[1]

    cr = jnp.transpose(coord, (0, 2, 1))
    cc = coord

    def flat(W):
        return jnp.transpose(W, (1, 0, 2)).reshape(W.shape[1], -1)

    full = lambda shp: pl.BlockSpec(shp, lambda s: (0,) * len(shp))
    cur = lambda s: jnp.minimum(s, B - 1)
    prev = lambda s: jnp.maximum(s - 1, 0)

    grid_spec = pl.GridSpec(
        grid=(B + 1,),
        in_specs=[
            pl.BlockSpec((1, 2, N), lambda s: (cur(s), 0, 0)),
            pl.BlockSpec((1, N, 2), lambda s: (cur(s), 0, 0)),
            pl.BlockSpec((1, N, N), lambda s: (cur(s), 0, 0)),
            pl.BlockSpec((1, N, Cin), lambda s: (prev(s), 0, 0)),
            full((Cin, KS * KS * C)), full((Cin, C)), full((1, C)),
            full((C, KS * KS * C)), full((C, C)), full((1, C)),
            full((C, KS * KS * C)), full((C, C)), full((1, C)),
            full((C, 10)), full((1, 10)),
        ],
        out_specs=[
            pl.BlockSpec((1, 1, 10), lambda s: (prev(s), 0, 0)),
            pl.BlockSpec((1, N, N), lambda s: (cur(s), 0, 0)),
            pl.BlockSpec((1, N, N), lambda s: (cur(s), 0, 0)),
        ],
        scratch_shapes=[
            pltpu.VMEM((2, KS * KS, N, N), jnp.bfloat16),
            pltpu.VMEM((2, N, 1), jnp.float32),
        ],
    )

    logits, u0, u1 = pl.pallas_call(
        _fused_body,
        grid_spec=grid_spec,
        out_shape=[
            jax.ShapeDtypeStruct((B, 1, 10), jnp.float32),
            jax.ShapeDtypeStruct((B, N, N), jnp.float32),
            jax.ShapeDtypeStruct((B, N, N), jnp.float32),
        ],
    )(cr, cc, A, x,
      flat(W1), R1, b1[None, :],
      flat(W2), R2, b2[None, :],
      flat(W3), R3, b3[None, :],
      Wfc, bfc[None, :])

    u = jnp.stack([u0, u1], axis=-1)
    return logits[:, 0, :], u


# speedup vs baseline: 1.1834x; 1.1834x over previous
import jax
import jax.numpy as jnp
import numpy as np
from jax.experimental import pallas as pl

KS = 3


def _fused_body(cr_ref, cc_ref, a_ref, x_ref,
                w1_ref, r1_ref, b1_ref,
                w2_ref, r2_ref, b2_ref,
                w3_ref, r3_ref, b3_ref,
                wfc_ref, bfc_ref,
                logits_ref, u0_ref, u1_ref):
    a = a_ref[0]
    cx_row = cr_ref[0, 0:1, :]
    cy_row = cr_ref[0, 1:2, :]
    cx_col = cc_ref[0, :, 0:1]
    cy_col = cc_ref[0, :, 1:2]

    dx = cx_row - cx_col
    dy = cy_row - cy_col
    rho = jnp.sqrt(dx * dx + dy * dy + 1e-12)
    theta = jnp.arctan2(dy, dx)
    mask = (a > 0).astype(jnp.float32)
    rho_max = jnp.max(rho * mask) + 1e-12
    u0 = (rho / rho_max) * mask
    u1 = (theta * np.float32(1.0 / (2.0 * np.pi)) + 0.5) * mask
    u0_ref[0] = u0
    u1_ref[0] = u1

    p0 = u0 * (KS - 1)
    p1 = u1 * (KS - 1)
    b0 = [jnp.maximum(0.0, 1.0 - jnp.abs(p0 - k)) for k in range(KS)]
    b1 = [jnp.maximum(0.0, 1.0 - jnp.abs(p1 - k)) for k in range(KS)]

    deg = jnp.sum(a, axis=1, keepdims=True)
    inv_deg = 1.0 / jnp.maximum(deg, 1.0)

    ab0 = [a * b0[k1] for k1 in range(KS)]
    ws = [(ab0[k1] * b1[k2]).astype(jnp.bfloat16)
          for k1 in range(KS) for k2 in range(KS)]

    def mm(p, q):
        return jax.lax.dot_general(
            p, q, (((1,), (0,)), ((), ())),
            preferred_element_type=jnp.float32,
            precision=jax.lax.Precision.DEFAULT)

    def layer(h, w_ref, r_ref, b_ref):
        y = mm(h, w_ref[...])
        out = mm(h, r_ref[...]) + b_ref[...]
        c = r_ref.shape[1]
        msg_a = jnp.zeros_like(out)
        msg_b = jnp.zeros_like(out)
        yb = y.astype(jnp.bfloat16)
        for kidx in range(KS * KS):
            m = mm(ws[kidx], yb[:, kidx * c:(kidx + 1) * c])
            if kidx % 2 == 0:
                msg_a = msg_a + m
            else:
                msg_b = msg_b + m
        return jax.nn.relu(out + (msg_a + msg_b) * inv_deg)

    h = layer(x_ref[0], w1_ref, r1_ref, b1_ref)
    h = layer(h, w2_ref, r2_ref, b2_ref)
    h = layer(h, w3_ref, r3_ref, b3_ref)

    g = jnp.max(h, axis=0, keepdims=True)
    logits_ref[0] = mm(g, wfc_ref[...]) + bfc_ref[...]


def kernel(x, A, coord, W1, R1, b1, W2, R2, b2, W3, R3, b3, Wfc, bfc):
    B, N, Cin = x.shape
    C = R1.shape[1]

    cr = jnp.transpose(coord, (0, 2, 1))
    cc = coord

    def flat(W):
        return jnp.transpose(W, (1, 0, 2)).reshape(W.shape[1], -1)

    full = lambda shp: pl.BlockSpec(shp, lambda s: (0,) * len(shp))

    grid_spec = pl.GridSpec(
        grid=(B,),
        in_specs=[
            pl.BlockSpec((1, 2, N), lambda s: (s, 0, 0)),
            pl.BlockSpec((1, N, 2), lambda s: (s, 0, 0)),
            pl.BlockSpec((1, N, N), lambda s: (s, 0, 0)),
            pl.BlockSpec((1, N, Cin), lambda s: (s, 0, 0)),
            full((Cin, KS * KS * C)), full((Cin, C)), full((1, C)),
            full((C, KS * KS * C)), full((C, C)), full((1, C)),
            full((C, KS * KS * C)), full((C, C)), full((1, C)),
            full((C, 10)), full((1, 10)),
        ],
        out_specs=[
            pl.BlockSpec((1, 1, 10), lambda s: (s, 0, 0)),
            pl.BlockSpec((1, N, N), lambda s: (s, 0, 0)),
            pl.BlockSpec((1, N, N), lambda s: (s, 0, 0)),
        ],
    )

    logits, u0, u1 = pl.pallas_call(
        _fused_body,
        grid_spec=grid_spec,
        out_shape=[
            jax.ShapeDtypeStruct((B, 1, 10), jnp.float32),
            jax.ShapeDtypeStruct((B, N, N), jnp.float32),
            jax.ShapeDtypeStruct((B, N, N), jnp.float32),
        ],
    )(cr, cc, A, x,
      flat(W1), R1, b1[None, :],
      flat(W2), R2, b2[None, :],
      flat(W3), R3, b3[None, :],
      Wfc, bfc[None, :])

    u = jnp.stack([u0, u1], axis=-1)
    return logits[:, 0, :], u


# speedup vs baseline: 1.2278x; 1.0375x over previous
import jax
import jax.numpy as jnp
import numpy as np
from jax.experimental import pallas as pl

KS = 3


def _atan2(y, x):
    ax = jnp.abs(x)
    ay = jnp.abs(y)
    z = jnp.minimum(ax, ay) / (jnp.maximum(ax, ay) + np.float32(1e-30))
    z2 = z * z
    p = np.float32(-0.01172120)
    p = p * z2 + np.float32(0.05265332)
    p = p * z2 + np.float32(-0.11643287)
    p = p * z2 + np.float32(0.19354346)
    p = p * z2 + np.float32(-0.33262347)
    p = p * z2 + np.float32(0.99997726)
    t = p * z
    t = jnp.where(ay > ax, np.float32(np.pi / 2) - t, t)
    t = jnp.where(x < 0, np.float32(np.pi) - t, t)
    return jnp.where(y < 0, -t, t)


def _fused_body(cr_ref, cc_ref, a_ref, x_ref,
                w1_ref, r1_ref, b1_ref,
                w2_ref, r2_ref, b2_ref,
                w3_ref, r3_ref, b3_ref,
                wfc_ref, bfc_ref,
                logits_ref, u0_ref, u1_ref):
    a = a_ref[0]
    cx_row = cr_ref[0, 0:1, :]
    cy_row = cr_ref[0, 1:2, :]
    cx_col = cc_ref[0, :, 0:1]
    cy_col = cc_ref[0, :, 1:2]

    dx = cx_row - cx_col
    dy = cy_row - cy_col
    rho = jnp.sqrt(dx * dx + dy * dy + 1e-12)
    theta = _atan2(dy, dx)
    mask = (a > 0).astype(jnp.float32)
    rho_max = jnp.max(rho * mask) + 1e-12
    u0 = (rho / rho_max) * mask
    u1 = (theta * np.float32(1.0 / (2.0 * np.pi)) + 0.5) * mask
    u0_ref[0] = u0
    u1_ref[0] = u1

    p0 = u0 * (KS - 1)
    p1 = u1 * (KS - 1)
    b0 = [jnp.maximum(0.0, 1.0 - jnp.abs(p0 - k)) for k in range(KS)]
    b1 = [jnp.maximum(0.0, 1.0 - jnp.abs(p1 - k)) for k in range(KS)]

    deg = jnp.sum(a, axis=1, keepdims=True)
    inv_deg = 1.0 / jnp.maximum(deg, 1.0)

    ab0 = [a * b0[k1] for k1 in range(KS)]
    ws = [(ab0[k1] * b1[k2]).astype(jnp.bfloat16)
          for k1 in range(KS) for k2 in range(KS)]

    def mm(p, q):
        return jax.lax.dot_general(
            p, q, (((1,), (0,)), ((), ())),
            preferred_element_type=jnp.float32,
            precision=jax.lax.Precision.DEFAULT)

    def layer(h, w_ref, r_ref, b_ref):
        y = mm(h, w_ref[...])
        out = mm(h, r_ref[...]) + b_ref[...]
        c = r_ref.shape[1]
        msg_a = jnp.zeros_like(out)
        msg_b = jnp.zeros_like(out)
        yb = y.astype(jnp.bfloat16)
        for kidx in range(KS * KS):
            m = mm(ws[kidx], yb[:, kidx * c:(kidx + 1) * c])
            if kidx % 2 == 0:
                msg_a = msg_a + m
            else:
                msg_b = msg_b + m
        return jax.nn.relu(out + (msg_a + msg_b) * inv_deg)

    h = layer(x_ref[0], w1_ref, r1_ref, b1_ref)
    h = layer(h, w2_ref, r2_ref, b2_ref)
    h = layer(h, w3_ref, r3_ref, b3_ref)

    g = jnp.max(h, axis=0, keepdims=True)
    logits_ref[0] = mm(g, wfc_ref[...]) + bfc_ref[...]


def kernel(x, A, coord, W1, R1, b1, W2, R2, b2, W3, R3, b3, Wfc, bfc):
    B, N, Cin = x.shape
    C = R1.shape[1]

    cr = jnp.transpose(coord, (0, 2, 1))
    cc = coord

    def flat(W):
        return jnp.transpose(W, (1, 0, 2)).reshape(W.shape[1], -1)

    full = lambda shp: pl.BlockSpec(shp, lambda s: (0,) * len(shp))

    grid_spec = pl.GridSpec(
        grid=(B,),
        in_specs=[
            pl.BlockSpec((1, 2, N), lambda s: (s, 0, 0)),
            pl.BlockSpec((1, N, 2), lambda s: (s, 0, 0)),
            pl.BlockSpec((1, N, N), lambda s: (s, 0, 0)),
            pl.BlockSpec((1, N, Cin), lambda s: (s, 0, 0)),
            full((Cin, KS * KS * C)), full((Cin, C)), full((1, C)),
            full((C, KS * KS * C)), full((C, C)), full((1, C)),
            full((C, KS * KS * C)), full((C, C)), full((1, C)),
            full((C, 10)), full((1, 10)),
        ],
        out_specs=[
            pl.BlockSpec((1, 1, 10), lambda s: (s, 0, 0)),
            pl.BlockSpec((1, N, N), lambda s: (s, 0, 0)),
            pl.BlockSpec((1, N, N), lambda s: (s, 0, 0)),
        ],
    )

    logits, u0, u1 = pl.pallas_call(
        _fused_body,
        grid_spec=grid_spec,
        out_shape=[
            jax.ShapeDtypeStruct((B, 1, 10), jnp.float32),
            jax.ShapeDtypeStruct((B, N, N), jnp.float32),
            jax.ShapeDtypeStruct((B, N, N), jnp.float32),
        ],
    )(cr, cc, A, x,
      flat(W1), R1, b1[None, :],
      flat(W2), R2, b2[None, :],
      flat(W3), R3, b3[None, :],
      Wfc, bfc[None, :])

    u = jnp.stack([u0, u1], axis=-1)
    return logits[:, 0, :], u
